# initial kernel scaffold (unmeasured)
import jax
import jax.numpy as jnp
from jax import lax
from jax.experimental import pallas as pl
from jax.experimental.pallas import tpu as pltpu

B, S, D = 4, 256, 4096
H, Dh, Dr = 32, 128, 64
M = B * S
NY = 4
SCALE = (Dh + Dr) ** -0.5


def _proj_kv_body(x_ref, wdkv_ref, wuk_ref, wuv_ref, out_ref):
    xb = x_ref[...].astype(jnp.bfloat16)
    c = jnp.dot(
        xb, wdkv_ref[...].astype(jnp.bfloat16),
        preferred_element_type=jnp.float32,
    ).astype(jnp.bfloat16)
    kp = jnp.dot(
        c, wuk_ref[...].astype(jnp.bfloat16),
        preferred_element_type=jnp.float32,
    )
    vp = jnp.dot(
        c, wuv_ref[...].astype(jnp.bfloat16),
        preferred_element_type=jnp.float32,
    )
    out_ref[0] = kp.astype(jnp.bfloat16)
    out_ref[1] = vp.astype(jnp.bfloat16)


def _proj_kv(x2d, wdkv, wuk, wuv):
    return pl.pallas_call(
        _proj_kv_body,
        out_shape=jax.ShapeDtypeStruct((2, M, D), jnp.bfloat16),
        in_specs=[pl.BlockSpec(memory_space=pltpu.VMEM)] * 4,
        out_specs=pl.BlockSpec(memory_space=pltpu.VMEM),
    )(x2d, wdkv, wuk, wuv)


def _allreduce_body(kv_ref, out_ref, comm_ref, send_sems, recv_sems):
    my_x = lax.axis_index("x")
    my_y = lax.axis_index("y")
    my_z = lax.axis_index("z")
    left = (my_y - 1) % NY
    right = (my_y + 1) % NY

    barrier = pltpu.get_barrier_semaphore()
    for nbr in (left, right):
        pl.semaphore_signal(
            barrier, inc=1,
            device_id=(my_x, nbr, my_z),
            device_id_type=pl.DeviceIdType.MESH,
        )
    pl.semaphore_wait(barrier, 2)

    out_ref[...] = kv_ref[...]
    comm_ref[0] = kv_ref[...]

    for h in range(NY - 1):
        s, r = h % 2, (h + 1) % 2
        rdma = pltpu.make_async_remote_copy(
            src_ref=comm_ref.at[s],
            dst_ref=comm_ref.at[r],
            send_sem=send_sems.at[s],
            recv_sem=recv_sems.at[r],
            device_id=(my_x, right, my_z),
            device_id_type=pl.DeviceIdType.MESH,
        )
        rdma.start()
        rdma.wait()
        out_ref[...] += comm_ref[r]


def _allreduce_y(kv):
    return pl.pallas_call(
        _allreduce_body,
        out_shape=jax.ShapeDtypeStruct((2, M, D), jnp.bfloat16),
        in_specs=[pl.BlockSpec(memory_space=pltpu.VMEM)],
        out_specs=pl.BlockSpec(memory_space=pltpu.VMEM),
        scratch_shapes=[
            pltpu.VMEM((2, 2, M, D), jnp.bfloat16),
            pltpu.SemaphoreType.DMA((2,)),
            pltpu.SemaphoreType.DMA((2,)),
        ],
        compiler_params=pltpu.CompilerParams(collective_id=0),
    )(kv)


def _matmul_body(a_ref, w_ref, out_ref):
    out_ref[...] = jnp.dot(
        a_ref[...].astype(jnp.bfloat16),
        w_ref[...].astype(jnp.bfloat16),
        preferred_element_type=jnp.float32,
    ).astype(out_ref.dtype)


def _matmul(a, w, n_block, out_dtype):
    m, k = a.shape
    _, n = w.shape
    return pl.pallas_call(
        _matmul_body,
        grid=(n // n_block,),
        in_specs=[
            pl.BlockSpec((m, k), lambda j: (0, 0)),
            pl.BlockSpec((k, n_block), lambda j: (0, j)),
        ],
        out_specs=pl.BlockSpec((m, n_block), lambda j: (0, j)),
        out_shape=jax.ShapeDtypeStruct((m, n), out_dtype),
    )(a, w)


def _attn_body(k_ref, v_ref, q_ref, qr_ref, kr_ref, o_ref):
    q = q_ref[...]
    k = k_ref[0]
    v = v_ref[0]
    s = lax.dot_general(
        q, k, (((1,), (1,)), ((), ())), preferred_element_type=jnp.float32
    )
    s += lax.dot_general(
        qr_ref[...], kr_ref[...], (((1,), (1,)), ((), ())),
        preferred_element_type=jnp.float32,
    )
    s *= SCALE
    mx = jnp.max(s, axis=-1, keepdims=True)
    p = jnp.exp(s - mx)
    p = p / jnp.sum(p, axis=-1, keepdims=True)
    o_ref[...] = jnp.dot(
        p.astype(jnp.bfloat16), v, preferred_element_type=jnp.float32
    ).astype(jnp.bfloat16)


def _attention(kv, q, qr, kr):
    return pl.pallas_call(
        _attn_body,
        grid=(B, H),
        in_specs=[
            pl.BlockSpec((1, S, Dh), lambda b, h: (0, b, h)),
            pl.BlockSpec((1, S, Dh), lambda b, h: (1, b, h)),
            pl.BlockSpec((S, Dh), lambda b, h: (b, h)),
            pl.BlockSpec((S, Dr), lambda b, h: (b, h)),
            pl.BlockSpec((S, Dr), lambda b, h: (b, 0)),
        ],
        out_specs=pl.BlockSpec((S, Dh), lambda b, h: (b, h)),
        out_shape=jax.ShapeDtypeStruct((M, H * Dh), jnp.bfloat16),
    )(kv, kv, q, qr, kr)


def kernel(x, Wdkv, Wuk, Wuv, Wq, Wqr, Wkr, Wo):
    x2d = x.reshape(M, D)

    kv_partial = _proj_kv(x2d, Wdkv, Wuk, Wuv)
    kv = _allreduce_y(kv_partial)

    q = _matmul(x2d, Wq, 512, jnp.bfloat16)
    qr = _matmul(x2d, Wqr, 512, jnp.bfloat16)
    kr = _matmul(x2d, Wkr, 64, jnp.bfloat16)

    o = _attention(kv, q, qr, kr)
    out = _matmul(o, Wo, 512, jnp.float32)
    return out.reshape(B, S, D)


# baseline (device time: 600969 ns/iter reference)
import jax
import jax.numpy as jnp
from jax import lax
from jax.experimental import pallas as pl
from jax.experimental.pallas import tpu as pltpu

B, S, D = 4, 256, 4096
H, Dh, Dr = 32, 128, 64
M = B * S
NY = 4
NCH = 4
CHM = 2 * M // NCH
SCALE = (Dh + Dr) ** -0.5
_VMEM_LIMIT = 100 * 1024 * 1024


def _matmul_body(a_ref, w_ref, out_ref):
    out_ref[...] = jnp.dot(
        a_ref[...].astype(jnp.bfloat16),
        w_ref[...].astype(jnp.bfloat16),
        preferred_element_type=jnp.float32,
    ).astype(out_ref.dtype)


def _matmul(a, w, n_block, out_dtype):
    m, k = a.shape
    _, n = w.shape
    return pl.pallas_call(
        _matmul_body,
        grid=(n // n_block,),
        in_specs=[
            pl.BlockSpec((m, k), lambda j: (0, 0)),
            pl.BlockSpec((k, n_block), lambda j: (0, j)),
        ],
        out_specs=pl.BlockSpec((m, n_block), lambda j: (0, j)),
        out_shape=jax.ShapeDtypeStruct((m, n), out_dtype),
        compiler_params=pltpu.CompilerParams(vmem_limit_bytes=_VMEM_LIMIT),
    )(a, w)


def _allreduce_body(kv_ref, out_ref, comm_ref, send_sems, recv_sems, credit_sem):
    out_ref[...] = kv_ref[...]
    my_x = lax.axis_index("x")
    my_y = lax.axis_index("y")
    my_z = lax.axis_index("z")
    left = (my_y + NY - 1) % NY
    right = (my_y + 1) % NY

    barrier = pltpu.get_barrier_semaphore()
    for nbr in (left, right):
        pl.semaphore_signal(
            barrier, inc=1,
            device_id=(my_x, nbr, my_z),
            device_id_type=pl.DeviceIdType.MESH,
        )
    pl.semaphore_wait(barrier, 2)

    for g in range(2 * (NY - 1)):
        slot = g % 2
        if g < NY - 1:
            t = g
            send_c = (my_y + NY - t) % NY
            recv_c = (my_y + NY - t - 1) % NY
            dst = comm_ref.at[slot]
        else:
            t = g - (NY - 1)
            send_c = (my_y + 1 + NY - t) % NY
            recv_c = (my_y + NY - t) % NY
            dst = out_ref.at[send_c]
        if g >= 2:
            pl.semaphore_wait(credit_sem, 1)
        rdma = pltpu.make_async_remote_copy(
            src_ref=out_ref.at[send_c],
            dst_ref=dst,
            send_sem=send_sems.at[slot],
            recv_sem=recv_sems.at[slot],
            device_id=(my_x, right, my_z),
            device_id_type=pl.DeviceIdType.MESH,
        )
        rdma.start()
        rdma.wait()
        if g < NY - 1:
            out_ref[recv_c] = out_ref[recv_c] + comm_ref[slot]
        if g < 2 * (NY - 1) - 2:
            pl.semaphore_signal(
                credit_sem, inc=1,
                device_id=(my_x, left, my_z),
                device_id_type=pl.DeviceIdType.MESH,
            )


def _allreduce_y(kv):
    return pl.pallas_call(
        _allreduce_body,
        out_shape=jax.ShapeDtypeStruct((NCH, CHM, D), jnp.bfloat16),
        in_specs=[pl.BlockSpec(memory_space=pltpu.VMEM)],
        out_specs=pl.BlockSpec(memory_space=pltpu.VMEM),
        scratch_shapes=[
            pltpu.VMEM((2, CHM, D), jnp.bfloat16),
            pltpu.SemaphoreType.DMA((2,)),
            pltpu.SemaphoreType.DMA((2,)),
            pltpu.SemaphoreType.REGULAR,
        ],
        input_output_aliases={0: 0},
        compiler_params=pltpu.CompilerParams(
            collective_id=0, vmem_limit_bytes=_VMEM_LIMIT
        ),
    )(kv)


def _attn_body(k_ref, v_ref, q_ref, qr_ref, kr_ref, o_ref):
    q = q_ref[...]
    k = k_ref[0]
    v = v_ref[0]
    s = lax.dot_general(
        q, k, (((1,), (1,)), ((), ())), preferred_element_type=jnp.float32
    )
    s += lax.dot_general(
        qr_ref[0, 0], kr_ref[...], (((1,), (1,)), ((), ())),
        preferred_element_type=jnp.float32,
    )
    s *= SCALE
    mx = jnp.max(s, axis=-1, keepdims=True)
    p = jnp.exp(s - mx)
    p = p / jnp.sum(p, axis=-1, keepdims=True)
    o_ref[...] = jnp.dot(
        p.astype(jnp.bfloat16), v, preferred_element_type=jnp.float32
    ).astype(jnp.bfloat16)


def _attention(kv, q, qr, kr):
    return pl.pallas_call(
        _attn_body,
        grid=(B, H),
        in_specs=[
            pl.BlockSpec((1, S, Dh), lambda b, h: (b // 2, b % 2, h)),
            pl.BlockSpec((1, S, Dh), lambda b, h: (2 + b // 2, b % 2, h)),
            pl.BlockSpec((S, Dh), lambda b, h: (b, h)),
            pl.BlockSpec((1, 1, S, Dr), lambda b, h: (b, h, 0, 0)),
            pl.BlockSpec((S, Dr), lambda b, h: (b, 0)),
        ],
        out_specs=pl.BlockSpec((S, Dh), lambda b, h: (b, h)),
        out_shape=jax.ShapeDtypeStruct((M, H * Dh), jnp.bfloat16),
        compiler_params=pltpu.CompilerParams(vmem_limit_bytes=_VMEM_LIMIT),
    )(kv, kv, q, qr, kr)


def kernel(x, Wdkv, Wuk, Wuv, Wq, Wqr, Wkr, Wo):
    x2d = x.reshape(M, D)

    c = _matmul(x2d, Wdkv, 128, jnp.bfloat16)
    kp = _matmul(c, Wuk, 512, jnp.bfloat16)
    vp = _matmul(c, Wuv, 512, jnp.bfloat16)
    kv_partial = jnp.concatenate([kp, vp], axis=0).reshape(NCH, CHM, D)
    kv = _allreduce_y(kv_partial)

    q = _matmul(x2d, Wq, 512, jnp.bfloat16)
    qr = _matmul(x2d, Wqr, 512, jnp.bfloat16)
    qr = qr.reshape(B, S, H, Dr).transpose(0, 2, 1, 3)
    kr = _matmul(x2d, Wkr, 64, jnp.bfloat16)

    o = _attention(kv, q, qr, kr)
    out = _matmul(o, Wo, 512, jnp.float32)
    return out.reshape(B, S, D)
